# baseline (speedup 1.0000x reference)
import functools
import math

import jax
import jax.numpy as jnp
from jax import lax
from jax.experimental import pallas as pl
from jax.experimental.pallas import tpu as pltpu
from jax.experimental.pallas import tpu_sc as plsc

NC = 2
NS = 16
NW = NC * NS
L = 16

D = 64
B = 4096 * 200
W = 128
NBUF = 8
K = 4
B_PER_W = B // NW
NCH = B_PER_W // W
SCALE = math.sqrt(D)

_mesh = plsc.VectorSubcoreMesh(core_axis_name="c", subcore_axis_name="s")


@functools.partial(
    pl.kernel,
    out_type=jax.ShapeDtypeStruct((B, D), jnp.float32),
    mesh=_mesh,
    scratch_types=[
        pltpu.VMEM((NCH, W), jnp.int32),
        pltpu.VMEM((NBUF, W, D), jnp.float32),
        pltpu.SemaphoreType.DMA((NBUF,)),
        pltpu.SemaphoreType.DMA((NBUF,)),
    ],
    compiler_params=pltpu.CompilerParams(use_tc_tiling_on_sc=False),
)
def _gather_scale(lut_hbm, xi_hbm, out_hbm, idx_v, rows, gsem, osem):
    wid = lax.axis_index("s") * NC + lax.axis_index("c")
    ch0 = wid * NCH
    pltpu.sync_copy(xi_hbm.at[pl.ds(ch0, NCH)], idx_v)

    for b in range(K):
        pltpu.make_async_copy(
            lut_hbm.at[idx_v.at[b]], rows.at[b], gsem.at[b]
        ).start()

    @pl.loop(0, NCH, step=NBUF)
    def _(c0):
        for bb in range(NBUF):
            c = c0 + bb
            pltpu.make_async_copy(
                lut_hbm.at[idx_v.at[c]], rows.at[bb], gsem.at[bb]
            ).wait()
            pltpu.make_async_copy(
                rows.at[bb], out_hbm.at[pl.ds((ch0 + c) * W, W)], osem.at[bb]
            ).start()
            b2 = (bb + K) % NBUF

            @pl.when(c + K < NCH)
            def _():
                @pl.when(c + K >= NBUF)
                def _():
                    pltpu.make_async_copy(
                        rows.at[b2],
                        out_hbm.at[pl.ds((ch0 + c + K - NBUF) * W, W)],
                        osem.at[b2],
                    ).wait()

                pltpu.make_async_copy(
                    lut_hbm.at[idx_v.at[c + K]], rows.at[b2], gsem.at[b2]
                ).start()

    for b in range(NBUF):
        pltpu.make_async_copy(
            rows.at[b],
            out_hbm.at[pl.ds((ch0 + NCH - NBUF + b) * W, W)],
            osem.at[b],
        ).wait()


def kernel(x, lut):
    xi = x.reshape(B // W, W).astype(jnp.int32)
    out = _gather_scale(lut, xi)
    return out.reshape(x.shape[0], x.shape[1], D)


# speedup vs baseline: 1.0523x; 1.0523x over previous
import functools
import math

import jax
import jax.numpy as jnp
from jax import lax
from jax.experimental import pallas as pl
from jax.experimental.pallas import tpu as pltpu
from jax.experimental.pallas import tpu_sc as plsc

NC = 2
NS = 16
NW = NC * NS
L = 16

V = 1000000
D = 64
DP = 128
B = 4096 * 200
SCALE = math.sqrt(D)

_mesh = plsc.VectorSubcoreMesh(core_axis_name="c", subcore_axis_name="s")

CHA = 160
NCHA = V // CHA
SLOTS = 3
KPT = -(-NCHA // NW)
NPJ = -(-KPT // SLOTS)


@functools.partial(
    pl.kernel,
    out_type=jax.ShapeDtypeStruct((V, DP), jnp.float32),
    mesh=_mesh,
    scratch_types=[
        pltpu.VMEM((SLOTS, CHA, D), jnp.float32),
        pltpu.VMEM((SLOTS, CHA, DP), jnp.float32),
        pltpu.SemaphoreType.DMA((SLOTS,)),
        pltpu.SemaphoreType.DMA((SLOTS,)),
    ],
    compiler_params=pltpu.CompilerParams(use_tc_tiling_on_sc=True),
)
def _relayout(lut_hbm, lutp_hbm, buf_r, buf_w, rsem, wsem):
    wid = lax.axis_index("s") * NC + lax.axis_index("c")
    kbase = wid * KPT

    for p in range(SLOTS):
        c = kbase + p

        @pl.when(c < NCHA)
        def _():
            pltpu.make_async_copy(
                lut_hbm.at[pl.ds(c * CHA, CHA)], buf_r.at[p], rsem.at[p]
            ).start()

    @pl.loop(0, NPJ)
    def _(j):
        for p in range(SLOTS):
            k = j * SLOTS + p
            c = kbase + k

            @pl.when((k < KPT) & (c < NCHA))
            def _():
                pltpu.make_async_copy(
                    lut_hbm.at[pl.ds(c * CHA, CHA)], buf_r.at[p], rsem.at[p]
                ).wait()

                @pl.when(k >= SLOTS)
                def _():
                    pltpu.make_async_copy(
                        buf_w.at[p],
                        lutp_hbm.at[pl.ds((c - SLOTS) * CHA, CHA)],
                        wsem.at[p],
                    ).wait()

                @pl.loop(0, CHA)
                def _(r):
                    for col in range(0, D, L):
                        buf_w.at[p, r, pl.ds(col, L)][...] = buf_r.at[
                            p, r, pl.ds(col, L)
                        ][...]

                cn = c + SLOTS

                @pl.when((k + SLOTS < KPT) & (cn < NCHA))
                def _():
                    pltpu.make_async_copy(
                        lut_hbm.at[pl.ds(cn * CHA, CHA)], buf_r.at[p], rsem.at[p]
                    ).start()

                pltpu.make_async_copy(
                    buf_w.at[p],
                    lutp_hbm.at[pl.ds(c * CHA, CHA)],
                    wsem.at[p],
                ).start()

    kmax = jnp.minimum(KPT, NCHA - kbase)
    for p in range(SLOTS):
        kl = kmax - 1 - ((kmax - 1 - p) % SLOTS)
        cl = kbase + kl

        @pl.when(kl >= 0)
        def _():
            pltpu.make_async_copy(
                buf_w.at[p],
                lutp_hbm.at[pl.ds(cl * CHA, CHA)],
                wsem.at[p],
            ).wait()


W = 128
NBUF = 4
NOB = 2
B_PER_W = B // NW
NCH = B_PER_W // W


@functools.partial(
    pl.kernel,
    out_type=jax.ShapeDtypeStruct((B, D), jnp.float32),
    mesh=_mesh,
    scratch_types=[
        pltpu.VMEM((NCH, W), jnp.int32),
        pltpu.VMEM((NBUF, W, DP), jnp.float32),
        pltpu.VMEM((NOB, W, D), jnp.float32),
        pltpu.SemaphoreType.DMA((NBUF,)),
        pltpu.SemaphoreType.DMA((NOB,)),
    ],
    compiler_params=pltpu.CompilerParams(use_tc_tiling_on_sc=True),
)
def _gather(lutp_hbm, xi_hbm, out_hbm, idx_v, rows, obuf, gsem, osem):
    wid = lax.axis_index("s") * NC + lax.axis_index("c")
    ch0 = wid * NCH
    pltpu.sync_copy(xi_hbm.at[pl.ds(ch0, NCH)], idx_v)

    for b in range(NBUF):
        pltpu.make_async_copy(
            lutp_hbm.at[idx_v.at[b]], rows.at[b], gsem.at[b]
        ).start()

    @pl.loop(0, NCH, step=NBUF)
    def _(c0):
        for bb in range(NBUF):
            c = c0 + bb
            q = bb % NOB
            pltpu.make_async_copy(
                lutp_hbm.at[idx_v.at[c]], rows.at[bb], gsem.at[bb]
            ).wait()

            def _drain():
                pltpu.make_async_copy(
                    obuf.at[q],
                    out_hbm.at[pl.ds((ch0 + c - NOB) * W, W)],
                    osem.at[q],
                ).wait()

            if bb >= NOB:
                _drain()
            else:
                pl.when(c0 > 0)(_drain)

            @pl.loop(0, W)
            def _(r):
                for col in range(0, D, L):
                    obuf.at[q, r, pl.ds(col, L)][...] = (
                        rows.at[bb, r, pl.ds(col, L)][...] * SCALE
                    )

            pltpu.make_async_copy(
                obuf.at[q], out_hbm.at[pl.ds((ch0 + c) * W, W)], osem.at[q]
            ).start()

            @pl.when(c + NBUF < NCH)
            def _():
                pltpu.make_async_copy(
                    lutp_hbm.at[idx_v.at[c + NBUF]], rows.at[bb], gsem.at[bb]
                ).start()

    for b in range(NOB):
        pltpu.make_async_copy(
            obuf.at[(NCH - NOB + b) % NOB],
            out_hbm.at[pl.ds((ch0 + NCH - NOB + b) * W, W)],
            osem.at[(NCH - NOB + b) % NOB],
        ).wait()


def kernel(x, lut):
    xi = x.reshape(B // W, W).astype(jnp.int32)
    lutp = _relayout(lut)
    out = _gather(lutp, xi)
    return out.reshape(x.shape[0], x.shape[1], D)
